# baseline (speedup 1.0000x reference)
import functools

import jax
import jax.numpy as jnp
from jax import lax
from jax.experimental import pallas as pl
from jax.experimental.pallas import tpu as pltpu
from jax.experimental.pallas import tpu_sc as plsc

N = 16384
IN_C = 128
OUT_C = 256
K = 16
M = 4096
EPS = 1e-5
PR = 128
PC = 128
TM = 128
RB = 1024

_DEF = jax.lax.Precision.DEFAULT


def _argmax_fold(v, i, x, y, z):
    def merge(parts):
        def halves(a):
            return (a[:h], a[h:]) if axis == 0 else (a[:, :h], a[:, h:])
        av, bv = halves(parts[0])
        ai, bi = halves(parts[1])
        pick = (av > bv) | ((av == bv) & (ai < bi))
        out = [jnp.where(pick, av, bv), jnp.where(pick, ai, bi)]
        for a in parts[2:]:
            aa, ba = halves(a)
            out.append(jnp.where(pick, aa, ba))
        return out

    parts = [v, i, x, y, z]
    while parts[0].shape[0] > 8:
        axis, h = 0, parts[0].shape[0] // 2
        parts = merge(parts)
    while parts[0].shape[1] > 1:
        axis, h = 1, parts[0].shape[1] // 2
        parts = merge(parts)
    while parts[0].shape[0] > 1:
        axis, h = 0, parts[0].shape[0] // 2
        parts = merge(parts)
    return parts


def _fps_body(px_ref, py_ref, pz_ref, sel_ref, sx_ref, sy_ref, sz_ref,
              dists_ref):
    row_iota = lax.broadcasted_iota(jnp.int32, (PR, PC), 0)
    col_iota = lax.broadcasted_iota(jnp.int32, (PR, PC), 1)
    flat = row_iota * PC + col_iota

    def step(qx, qy, qz, prev):
        px = px_ref[...]
        py = py_ref[...]
        pz = pz_ref[...]
        dx = px - qx
        dy = py - qy
        dz = pz - qz
        d = (dx * dx + dz * dz) + dy * dy
        dmin = d if prev is None else jnp.minimum(prev, d)
        dists_ref[...] = dmin
        return _argmax_fold(dmin, flat, px, py, pz)

    px00 = px_ref[0:1, 0:1]
    py00 = py_ref[0:1, 0:1]
    pz00 = pz_ref[0:1, 0:1]
    sel_ref[0] = jnp.int32(0)
    sx_ref[0] = px00[0, 0]
    sy_ref[0] = py00[0, 0]
    sz_ref[0] = pz00[0, 0]
    _, idx0, qx0, qy0, qz0 = step(px00, py00, pz00, None)

    def body(m, carry):
        idx, qx, qy, qz = carry
        sel_ref[m] = idx[0, 0]
        sx_ref[m] = qx[0, 0]
        sy_ref[m] = qy[0, 0]
        sz_ref[m] = qz[0, 0]
        _, nidx, nqx, nqy, nqz = step(qx, qy, qz, dists_ref[...])
        return (nidx, nqx, nqy, nqz)

    lax.fori_loop(1, M, body, (idx0, qx0, qy0, qz0))


def _run_fps(px, py, pz):
    smem = pl.BlockSpec(memory_space=pltpu.SMEM)
    return pl.pallas_call(
        _fps_body,
        out_shape=(
            jax.ShapeDtypeStruct((M,), jnp.int32),
            jax.ShapeDtypeStruct((M,), jnp.float32),
            jax.ShapeDtypeStruct((M,), jnp.float32),
            jax.ShapeDtypeStruct((M,), jnp.float32),
        ),
        out_specs=(smem, smem, smem, smem),
        scratch_shapes=[pltpu.VMEM((PR, PC), jnp.float32)],
    )(px, py, pz)


def _knn_body(q_ref, pos_ref, nbr_ref, d2_ref):
    a = jnp.dot(q_ref[...], pos_ref[...], precision=_DEF,
                preferred_element_type=jnp.float32)
    qx = q_ref[:, 0:1]
    qy = q_ref[:, 1:2]
    qz = q_ref[:, 2:3]
    qq = (qx * qx + qz * qz) + qy * qy
    px = pos_ref[0:1, :]
    py = pos_ref[1:2, :]
    pz = pos_ref[2:3, :]
    pp = (px * px + pz * pz) + py * py
    d2_ref[...] = (qq - 2.0 * a) + pp
    col = lax.broadcasted_iota(jnp.int32, (TM, N), 1)
    idx = None
    for k in range(K):
        d2 = d2_ref[...]
        if k > 0:
            d2 = jnp.where(col == idx, jnp.float32(jnp.inf), d2)
            d2_ref[...] = d2
        mv = jnp.min(d2, axis=1, keepdims=True)
        idx = jnp.min(jnp.where(d2 == mv, col, jnp.int32(N)),
                      axis=1, keepdims=True)
        nbr_ref[:, pl.ds(k, 1)] = idx


def _run_knn(qpad, pospadT):
    return pl.pallas_call(
        _knn_body,
        grid=(M // TM,),
        in_specs=[
            pl.BlockSpec((TM, 128), lambda i: (i, 0)),
            pl.BlockSpec((128, N), lambda i: (0, 0)),
        ],
        out_specs=pl.BlockSpec((TM, K), lambda i: (i, 0)),
        out_shape=jax.ShapeDtypeStruct((M, K), jnp.int32),
        scratch_shapes=[pltpu.VMEM((TM, N), jnp.float32)],
    )(qpad, pospadT)


def _mm_body(x_ref, w_ref, b_ref, g_ref, s_ref, ss_ref):
    i = pl.program_id(0)
    g = jnp.dot(x_ref[...], w_ref[...], precision=_DEF,
                preferred_element_type=jnp.float32) + b_ref[...]
    g_ref[...] = g
    ps = jnp.sum(g, axis=0, keepdims=True)
    pss = jnp.sum(g * g, axis=0, keepdims=True)

    @pl.when(i == 0)
    def _init():
        s_ref[...] = ps
        ss_ref[...] = pss

    @pl.when(i > 0)
    def _acc():
        s_ref[...] += ps
        ss_ref[...] += pss


def _run_mm(x, W, b2d):
    return pl.pallas_call(
        _mm_body,
        grid=(N // RB,),
        in_specs=[
            pl.BlockSpec((RB, IN_C), lambda i: (i, 0)),
            pl.BlockSpec((IN_C, OUT_C), lambda i: (0, 0)),
            pl.BlockSpec((1, OUT_C), lambda i: (0, 0)),
        ],
        out_specs=(
            pl.BlockSpec((RB, OUT_C), lambda i: (i, 0)),
            pl.BlockSpec((1, OUT_C), lambda i: (0, 0)),
            pl.BlockSpec((1, OUT_C), lambda i: (0, 0)),
        ),
        out_shape=(
            jax.ShapeDtypeStruct((N, OUT_C), jnp.float32),
            jax.ShapeDtypeStruct((1, OUT_C), jnp.float32),
            jax.ShapeDtypeStruct((1, OUT_C), jnp.float32),
        ),
    )(x, W, b2d)


def _stats2_body(g_ref, a1_ref, c1_ref, s_ref, ss_ref):
    i = pl.program_id(0)
    h = jnp.maximum(a1_ref[...] * g_ref[...] + c1_ref[...], 0.0)
    ps = jnp.sum(h, axis=0, keepdims=True)
    pss = jnp.sum(h * h, axis=0, keepdims=True)

    @pl.when(i == 0)
    def _init():
        s_ref[...] = ps
        ss_ref[...] = pss

    @pl.when(i > 0)
    def _acc():
        s_ref[...] += ps
        ss_ref[...] += pss


def _run_stats2(g, a1, c1):
    return pl.pallas_call(
        _stats2_body,
        grid=(N // RB,),
        in_specs=[
            pl.BlockSpec((RB, OUT_C), lambda i: (i, 0)),
            pl.BlockSpec((1, OUT_C), lambda i: (0, 0)),
            pl.BlockSpec((1, OUT_C), lambda i: (0, 0)),
        ],
        out_specs=(
            pl.BlockSpec((1, OUT_C), lambda i: (0, 0)),
            pl.BlockSpec((1, OUT_C), lambda i: (0, 0)),
        ),
        out_shape=(
            jax.ShapeDtypeStruct((1, OUT_C), jnp.float32),
            jax.ShapeDtypeStruct((1, OUT_C), jnp.float32),
        ),
    )(g, a1, c1)


_NC = 2
_NS = 16
_NW = _NC * _NS
_MC = M // _NW
_CC = 4
_NCH = _MC // _CC
_L = 16


def _segmax_body(g_hbm, nbr_hbm, par_hbm, out_hbm, idx_v, rows_v, outb_v,
                 par_v, sem):
    wid = lax.axis_index("s") * _NC + lax.axis_index("c")
    pltpu.sync_copy(par_hbm, par_v)

    def chunk(ci, carry):
        base = wid * _MC + ci * _CC
        pltpu.sync_copy(nbr_hbm.at[pl.ds(base * K, _CC * K)], idx_v)
        pltpu.async_copy(g_hbm.at[idx_v], rows_v, sem).wait()
        for j in range(_CC):
            for cb in range(OUT_C // _L):
                sl = pl.ds(cb * _L, _L)
                acc = rows_v[j * K, sl]
                for k in range(1, K):
                    acc = jnp.maximum(acc, rows_v[j * K + k, sl])
                t = jnp.maximum(par_v[0, sl] * acc + par_v[1, sl], 0.0)
                u = par_v[2, sl] * t + par_v[3, sl]
                outb_v[j, sl] = jnp.where(u >= 0.0, u, 0.01 * u)
        pltpu.sync_copy(outb_v, out_hbm.at[pl.ds(base, _CC)])
        return carry

    lax.fori_loop(0, _NCH, chunk, 0)


@functools.partial(
    pl.kernel,
    out_type=jax.ShapeDtypeStruct((M, OUT_C), jnp.float32),
    mesh=plsc.VectorSubcoreMesh(core_axis_name="c", subcore_axis_name="s"),
    scratch_types=[
        pltpu.VMEM((_CC * K,), jnp.int32),
        pltpu.VMEM((_CC * K, OUT_C), jnp.float32),
        pltpu.VMEM((_CC, OUT_C), jnp.float32),
        pltpu.VMEM((4, OUT_C), jnp.float32),
        pltpu.SemaphoreType.DMA,
    ],
)
def _run_segmax(g_hbm, nbr_hbm, par_hbm, out_hbm, idx_v, rows_v, outb_v,
                par_v, sem):
    _segmax_body(g_hbm, nbr_hbm, par_hbm, out_hbm, idx_v, rows_v, outb_v,
                 par_v, sem)


def kernel(x, pos, batch, W, b, gamma1, beta1, gamma2, beta2):
    px = pos[:, 0].reshape(PR, PC)
    py = pos[:, 1].reshape(PR, PC)
    pz = pos[:, 2].reshape(PR, PC)

    sel, sx, sy, sz = _run_fps(px, py, pz)
    subpos = jnp.stack((sx, sy, sz), axis=-1)

    qpad = jnp.zeros((M, 128), jnp.float32).at[:, :3].set(subpos)
    pospadT = jnp.zeros((128, N), jnp.float32).at[:3, :].set(pos.T)
    nbr = _run_knn(qpad, pospadT)

    g, s1, ss1 = _run_mm(x, W, b.reshape(1, OUT_C))
    mu1 = s1 / N
    var1 = ss1 / N - mu1 * mu1
    a1 = gamma1.reshape(1, OUT_C) / jnp.sqrt(var1 + EPS)
    c1 = beta1.reshape(1, OUT_C) - mu1 * a1

    s2, ss2 = _run_stats2(g, a1, c1)
    mu2 = s2 / N
    var2 = ss2 / N - mu2 * mu2
    a2 = gamma2.reshape(1, OUT_C) / jnp.sqrt(var2 + EPS)
    c2 = beta2.reshape(1, OUT_C) - mu2 * a2

    params = jnp.concatenate([a1, c1, a2, c2], axis=0)
    out = _run_segmax(g, nbr.reshape(M * K), params)

    return (out, subpos, batch[sel])


# speedup vs baseline: 1.1214x; 1.1214x over previous
import functools

import jax
import jax.numpy as jnp
from jax import lax
from jax.experimental import pallas as pl
from jax.experimental.pallas import tpu as pltpu
from jax.experimental.pallas import tpu_sc as plsc

N = 16384
IN_C = 128
OUT_C = 256
K = 16
M = 4096
EPS = 1e-5
PR = 128
PC = 128
TM = 128
RB = 1024

_DEF = jax.lax.Precision.DEFAULT


def _fps_body(px_ref, py_ref, pz_ref, sel_ref, sx_ref, sy_ref, sz_ref,
              dists_ref):
    col1 = lax.broadcasted_iota(jnp.int32, (1, PC), 1)
    row_iota = lax.broadcasted_iota(jnp.int32, (PR, PC), 0)
    col_iota = lax.broadcasted_iota(jnp.int32, (PR, PC), 1)
    flat = row_iota * PC + col_iota

    def extract(r, c):
        m = col1 == c
        qx = jnp.sum(jnp.where(m, px_ref[pl.ds(r, 1), :], 0.0))
        qy = jnp.sum(jnp.where(m, py_ref[pl.ds(r, 1), :], 0.0))
        qz = jnp.sum(jnp.where(m, pz_ref[pl.ds(r, 1), :], 0.0))
        return qx, qy, qz

    def dist(qx, qy, qz):
        dx = px_ref[...] - qx
        dy = py_ref[...] - qy
        dz = pz_ref[...] - qz
        return (dx * dx + dz * dz) + dy * dy

    qx, qy, qz = extract(0, 0)
    sel_ref[0] = jnp.int32(0)
    sx_ref[0] = qx
    sy_ref[0] = qy
    sz_ref[0] = qz
    d0 = dist(qx, qy, qz)
    dists_ref[...] = d0
    mx0 = jnp.max(d0)
    idx0 = jnp.min(jnp.where(d0 == mx0, flat, jnp.int32(N)))

    def body(m, idx):
        sel_ref[m] = idx
        r = idx // PC
        c = idx - r * PC
        qx, qy, qz = extract(r, c)
        sx_ref[m] = qx
        sy_ref[m] = qy
        sz_ref[m] = qz
        dmin = jnp.minimum(dists_ref[...], dist(qx, qy, qz))
        dists_ref[...] = dmin
        mx = jnp.max(dmin)
        return jnp.min(jnp.where(dmin == mx, flat, jnp.int32(N)))

    lax.fori_loop(1, M, body, idx0)


def _run_fps(px, py, pz):
    smem = pl.BlockSpec(memory_space=pltpu.SMEM)
    return pl.pallas_call(
        _fps_body,
        out_shape=(
            jax.ShapeDtypeStruct((M,), jnp.int32),
            jax.ShapeDtypeStruct((M,), jnp.float32),
            jax.ShapeDtypeStruct((M,), jnp.float32),
            jax.ShapeDtypeStruct((M,), jnp.float32),
        ),
        out_specs=(smem, smem, smem, smem),
        scratch_shapes=[pltpu.VMEM((PR, PC), jnp.float32)],
    )(px, py, pz)


def _knn_body(q_ref, pos_ref, nbr_ref, d2_ref):
    a = jnp.dot(q_ref[...], pos_ref[...], precision=_DEF,
                preferred_element_type=jnp.float32)
    qx = q_ref[:, 0:1]
    qy = q_ref[:, 1:2]
    qz = q_ref[:, 2:3]
    qq = (qx * qx + qz * qz) + qy * qy
    px = pos_ref[0:1, :]
    py = pos_ref[1:2, :]
    pz = pos_ref[2:3, :]
    pp = (px * px + pz * pz) + py * py
    d2_ref[...] = (qq - 2.0 * a) + pp
    col = lax.broadcasted_iota(jnp.int32, (TM, N), 1)
    idx = None
    for k in range(K):
        d2 = d2_ref[...]
        if k > 0:
            d2 = jnp.where(col == idx, jnp.float32(jnp.inf), d2)
            d2_ref[...] = d2
        mv = jnp.min(d2, axis=1, keepdims=True)
        idx = jnp.min(jnp.where(d2 == mv, col, jnp.int32(N)),
                      axis=1, keepdims=True)
        nbr_ref[:, pl.ds(k, 1)] = idx


def _run_knn(qpad, pospadT):
    return pl.pallas_call(
        _knn_body,
        grid=(M // TM,),
        in_specs=[
            pl.BlockSpec((TM, 128), lambda i: (i, 0)),
            pl.BlockSpec((128, N), lambda i: (0, 0)),
        ],
        out_specs=pl.BlockSpec((TM, K), lambda i: (i, 0)),
        out_shape=jax.ShapeDtypeStruct((M, K), jnp.int32),
        scratch_shapes=[pltpu.VMEM((TM, N), jnp.float32)],
    )(qpad, pospadT)


def _mm_body(x_ref, w_ref, b_ref, g_ref, s_ref, ss_ref):
    i = pl.program_id(0)
    g = jnp.dot(x_ref[...], w_ref[...], precision=_DEF,
                preferred_element_type=jnp.float32) + b_ref[...]
    g_ref[...] = g
    ps = jnp.sum(g, axis=0, keepdims=True)
    pss = jnp.sum(g * g, axis=0, keepdims=True)

    @pl.when(i == 0)
    def _init():
        s_ref[...] = ps
        ss_ref[...] = pss

    @pl.when(i > 0)
    def _acc():
        s_ref[...] += ps
        ss_ref[...] += pss


def _run_mm(x, W, b2d):
    return pl.pallas_call(
        _mm_body,
        grid=(N // RB,),
        in_specs=[
            pl.BlockSpec((RB, IN_C), lambda i: (i, 0)),
            pl.BlockSpec((IN_C, OUT_C), lambda i: (0, 0)),
            pl.BlockSpec((1, OUT_C), lambda i: (0, 0)),
        ],
        out_specs=(
            pl.BlockSpec((RB, OUT_C), lambda i: (i, 0)),
            pl.BlockSpec((1, OUT_C), lambda i: (0, 0)),
            pl.BlockSpec((1, OUT_C), lambda i: (0, 0)),
        ),
        out_shape=(
            jax.ShapeDtypeStruct((N, OUT_C), jnp.float32),
            jax.ShapeDtypeStruct((1, OUT_C), jnp.float32),
            jax.ShapeDtypeStruct((1, OUT_C), jnp.float32),
        ),
    )(x, W, b2d)


def _stats2_body(g_ref, a1_ref, c1_ref, s_ref, ss_ref):
    i = pl.program_id(0)
    h = jnp.maximum(a1_ref[...] * g_ref[...] + c1_ref[...], 0.0)
    ps = jnp.sum(h, axis=0, keepdims=True)
    pss = jnp.sum(h * h, axis=0, keepdims=True)

    @pl.when(i == 0)
    def _init():
        s_ref[...] = ps
        ss_ref[...] = pss

    @pl.when(i > 0)
    def _acc():
        s_ref[...] += ps
        ss_ref[...] += pss


def _run_stats2(g, a1, c1):
    return pl.pallas_call(
        _stats2_body,
        grid=(N // RB,),
        in_specs=[
            pl.BlockSpec((RB, OUT_C), lambda i: (i, 0)),
            pl.BlockSpec((1, OUT_C), lambda i: (0, 0)),
            pl.BlockSpec((1, OUT_C), lambda i: (0, 0)),
        ],
        out_specs=(
            pl.BlockSpec((1, OUT_C), lambda i: (0, 0)),
            pl.BlockSpec((1, OUT_C), lambda i: (0, 0)),
        ),
        out_shape=(
            jax.ShapeDtypeStruct((1, OUT_C), jnp.float32),
            jax.ShapeDtypeStruct((1, OUT_C), jnp.float32),
        ),
    )(g, a1, c1)


_NC = 2
_NS = 16
_NW = _NC * _NS
_MC = M // _NW
_CC = 8
_NCH = _MC // _CC
_L = 16


def _segmax_body(g_hbm, nbr_hbm, par_hbm, out_hbm, idx_v, rows_v, outb_v,
                 par_v, sem):
    wid = lax.axis_index("s") * _NC + lax.axis_index("c")
    pltpu.sync_copy(par_hbm, par_v)

    def chunk(ci, carry):
        base = wid * _MC + ci * _CC
        pltpu.sync_copy(nbr_hbm.at[pl.ds(base * K, _CC * K)], idx_v)
        pltpu.async_copy(g_hbm.at[idx_v], rows_v, sem).wait()
        for j in range(_CC):
            for cb in range(OUT_C // _L):
                sl = pl.ds(cb * _L, _L)
                acc = rows_v[j * K, sl]
                for k in range(1, K):
                    acc = jnp.maximum(acc, rows_v[j * K + k, sl])
                t = jnp.maximum(par_v[0, sl] * acc + par_v[1, sl], 0.0)
                u = par_v[2, sl] * t + par_v[3, sl]
                outb_v[j, sl] = jnp.where(u >= 0.0, u, 0.01 * u)
        pltpu.sync_copy(outb_v, out_hbm.at[pl.ds(base, _CC)])
        return carry

    lax.fori_loop(0, _NCH, chunk, 0)


@functools.partial(
    pl.kernel,
    out_type=jax.ShapeDtypeStruct((M, OUT_C), jnp.float32),
    mesh=plsc.VectorSubcoreMesh(core_axis_name="c", subcore_axis_name="s"),
    scratch_types=[
        pltpu.VMEM((_CC * K,), jnp.int32),
        pltpu.VMEM((_CC * K, OUT_C), jnp.float32),
        pltpu.VMEM((_CC, OUT_C), jnp.float32),
        pltpu.VMEM((4, OUT_C), jnp.float32),
        pltpu.SemaphoreType.DMA,
    ],
)
def _run_segmax(g_hbm, nbr_hbm, par_hbm, out_hbm, idx_v, rows_v, outb_v,
                par_v, sem):
    _segmax_body(g_hbm, nbr_hbm, par_hbm, out_hbm, idx_v, rows_v, outb_v,
                 par_v, sem)


def kernel(x, pos, batch, W, b, gamma1, beta1, gamma2, beta2):
    px = pos[:, 0].reshape(PR, PC)
    py = pos[:, 1].reshape(PR, PC)
    pz = pos[:, 2].reshape(PR, PC)

    sel, sx, sy, sz = _run_fps(px, py, pz)
    subpos = jnp.stack((sx, sy, sz), axis=-1)

    qpad = jnp.zeros((M, 128), jnp.float32).at[:, :3].set(subpos)
    pospadT = jnp.zeros((128, N), jnp.float32).at[:3, :].set(pos.T)
    nbr = _run_knn(qpad, pospadT)

    g, s1, ss1 = _run_mm(x, W, b.reshape(1, OUT_C))
    mu1 = s1 / N
    var1 = ss1 / N - mu1 * mu1
    a1 = gamma1.reshape(1, OUT_C) / jnp.sqrt(var1 + EPS)
    c1 = beta1.reshape(1, OUT_C) - mu1 * a1

    s2, ss2 = _run_stats2(g, a1, c1)
    mu2 = s2 / N
    var2 = ss2 / N - mu2 * mu2
    a2 = gamma2.reshape(1, OUT_C) / jnp.sqrt(var2 + EPS)
    c2 = beta2.reshape(1, OUT_C) - mu2 * a2

    params = jnp.concatenate([a1, c1, a2, c2], axis=0)
    out = _run_segmax(g, nbr.reshape(M * K), params)

    return (out, subpos, batch[sel])
